# Initial kernel scaffold; baseline (speedup 1.0000x reference)
import jax
import jax.numpy as jnp
from jax.experimental import pallas as pl
from jax.experimental.pallas import tpu as pltpu
from jax.experimental.pallas import tpu_sc as plsc

_LANES = 128
_BN = 256


def _argmin_tc_kernel(z_ref, cbt_ref, idx_ref):
    z = z_ref[...]
    cbt = cbt_ref[...]
    cnorm = jnp.sum(cbt * cbt, axis=0, keepdims=True)
    prod = jax.lax.dot_general(
        z, cbt, (((1,), (0,)), ((), ())),
        preferred_element_type=jnp.float32,
        precision=jax.lax.Precision.HIGHEST,
    )
    s = cnorm - 2.0 * prod
    bn, k = s.shape
    c = k // _LANES
    s3 = s.reshape(bn, c, _LANES)
    m = jnp.min(s3, axis=1)
    ci = jax.lax.broadcasted_iota(jnp.int32, (bn, c, _LANES), 1)
    cmin = jnp.min(jnp.where(s3 == m[:, None, :], ci, c), axis=1)
    li = jax.lax.broadcasted_iota(jnp.int32, (bn, _LANES), 1)
    kk = cmin * _LANES + li
    gm = jnp.min(m, axis=1, keepdims=True)
    idx = jnp.min(jnp.where(m == gm, kk, k), axis=1)
    idx_ref[...] = idx[None, :]


def _compute_indices(z_flat, cbt):
    n, d = z_flat.shape
    k = cbt.shape[1]
    grid = (n // _BN,)
    return pl.pallas_call(
        _argmin_tc_kernel,
        grid=grid,
        in_specs=[
            pl.BlockSpec((_BN, d), lambda i: (i, 0)),
            pl.BlockSpec((d, k), lambda i: (0, 0)),
        ],
        out_specs=pl.BlockSpec((1, _BN), lambda i: (0, i)),
        out_shape=jax.ShapeDtypeStruct((1, n), jnp.int32),
        compiler_params=pltpu.CompilerParams(
            dimension_semantics=("arbitrary",),
        ),
    )(z_flat, cbt).reshape(n)


def kernel(z_e, codebook):
    b, l, d = z_e.shape
    z_flat = z_e.reshape(b * l, d)
    indices = _compute_indices(z_flat, codebook.T)
    z_q = jnp.take(codebook, indices, axis=0).reshape(z_e.shape)
    return z_q, indices.reshape(b, l)


# speedup vs baseline: 1.3637x; 1.3637x over previous
import jax
import jax.numpy as jnp
from jax.experimental import pallas as pl
from jax.experimental.pallas import tpu as pltpu
from jax.experimental.pallas import tpu_sc as plsc

_BN = 1024


def _round_bf16_vals(x):
    u = jax.lax.bitcast_convert_type(x, jnp.uint32)
    u = u + jnp.uint32(0x7FFF) + ((u >> 16) & jnp.uint32(1))
    return jax.lax.bitcast_convert_type(u & jnp.uint32(0xFFFF0000), jnp.float32)


def _argmin_tc_kernel(cb_ref, zt_ref, zn_ref, cn_ref, idx_ref):
    cbf = _round_bf16_vals(cb_ref[...])
    zt = zt_ref[...]
    k = cbf.shape[0]
    bn = zt.shape[1]
    zb = (2.0 * zt).astype(jnp.bfloat16)
    convT = jax.lax.dot_general(
        cbf, zb, (((1,), (0,)), ((), ())),
        preferred_element_type=jnp.float32,
    )
    a = zn_ref[...]
    cn = cn_ref[...].reshape(k, 1)
    d_ = (a - convT) + cn
    gm = jnp.min(d_, axis=0, keepdims=True)
    ii = jax.lax.broadcasted_iota(jnp.int32, (k, bn), 0)
    idx = jnp.min(jnp.where(d_ == gm, ii, k), axis=0)
    idx_ref[...] = idx[None, :]


def _compute_indices(codebook, z_t, znorm, cnorm):
    k, d = codebook.shape
    n = z_t.shape[1]
    return pl.pallas_call(
        _argmin_tc_kernel,
        grid=(n // _BN,),
        in_specs=[
            pl.BlockSpec((k, d), lambda i: (0, 0)),
            pl.BlockSpec((d, _BN), lambda i: (0, i)),
            pl.BlockSpec((1, _BN), lambda i: (0, i)),
            pl.BlockSpec((1, k), lambda i: (0, 0)),
        ],
        out_specs=pl.BlockSpec((1, _BN), lambda i: (0, i)),
        out_shape=jax.ShapeDtypeStruct((1, n), jnp.int32),
        compiler_params=pltpu.CompilerParams(
            dimension_semantics=("arbitrary",),
        ),
    )(codebook, z_t, znorm, cnorm).reshape(n)


def _gather_codes_sc(codebook_padded, indices):
    n = indices.shape[0]
    d = codebook_padded.shape[1]
    w = 128
    mesh = plsc.VectorSubcoreMesh(core_axis_name="c", subcore_axis_name="s")

    @jax.jit
    def gather(cb, idx2d):
        @pl.kernel(out_type=jax.ShapeDtypeStruct((n, d), cb.dtype), mesh=mesh)
        def body(cb_hbm, i_hbm, o_hbm):
            def inner(i_vmem, o_vmem):
                pltpu.sync_copy(cb_hbm.at[i_vmem.at[0]], o_vmem)

            pltpu.emit_pipeline(
                inner,
                grid=(n // w,),
                in_specs=[pl.BlockSpec((1, w), index_map=lambda i: (0, i))],
                out_specs=[pl.BlockSpec((w, d), index_map=lambda i: (i, 0))],
                core_axis_name=("c", "s"),
                dimension_semantics=(pltpu.PARALLEL,),
            )(i_hbm, o_hbm)

        return body(cb, idx2d)

    return gather(codebook_padded, indices.reshape(1, n))


def kernel(z_e, codebook):
    b, l, d = z_e.shape
    n = b * l
    flat = z_e.reshape(n, d)
    znorm = jnp.sum(flat * flat, axis=-1)[None, :]
    cnorm = jnp.sum(codebook * codebook, axis=-1)[None, :]
    indices = _compute_indices(codebook, flat.T, znorm, cnorm)
    cb_pad = jnp.pad(codebook, ((0, 0), (0, 128 - d)))
    z_q = _gather_codes_sc(cb_pad, indices)[:, :d].reshape(z_e.shape)
    return z_q, indices.reshape(b, l)
